# Initial kernel scaffold; baseline (speedup 1.0000x reference)
import jax
import jax.numpy as jnp
from jax.experimental import pallas as pl


def kernel(x, edge_index, W1, b1, W2, b2, W3, b3, Wd1, bd1, Wd2, bd2, Wo, bo):
    raise NotImplementedError("write your pallas kernel here")


# speedup vs baseline: 13.0094x; 13.0094x over previous
import functools

import jax
import jax.numpy as jnp
from jax import lax
from jax.experimental import pallas as pl
from jax.experimental.pallas import tpu as pltpu
from jax.experimental.pallas import tpu_sc as plsc

N = 10000
NP = 10240
D = 128
E = 320000
NC, NS = 2, 16
NW = NC * NS
CH = 128
EP = NW * 80 * CH
EPW = EP // NW
NCHUNK = EPW // CH

_mesh = plsc.VectorSubcoreMesh(
    core_axis_name="c", subcore_axis_name="s", num_cores=NC, num_subcores=NS)

ROWS_PER_TILE = NP // NS


@functools.partial(
    pl.kernel,
    out_type=jax.ShapeDtypeStruct((NC, NP, 1), jnp.float32),
    mesh=_mesh,
    scratch_types=[
        pltpu.VMEM((CH,), jnp.int32),
        pltpu.VMEM((CH, 1), jnp.float32),
        pltpu.VMEM_SHARED((NP, 1), jnp.float32),
    ],
)
def _sc_degree(dst_hbm, ones_hbm, zeros_hbm, out_hbm, didx, ones_v, deg_sh):
    c = lax.axis_index("c")
    s = lax.axis_index("s")
    wid = s * NC + c
    ebase = wid * EPW
    rbase = s * ROWS_PER_TILE

    pltpu.sync_copy(ones_hbm, ones_v)
    pltpu.sync_copy(zeros_hbm, deg_sh.at[pl.ds(rbase, ROWS_PER_TILE)])
    plsc.subcore_barrier()

    def step(i, carry):
        pltpu.sync_copy(dst_hbm.at[pl.ds(ebase + i * CH, CH)], didx)
        pltpu.sync_copy(ones_v, deg_sh.at[didx], add=True)
        return carry

    lax.fori_loop(0, NCHUNK, step, 0)
    plsc.subcore_barrier()
    pltpu.sync_copy(deg_sh.at[pl.ds(rbase, ROWS_PER_TILE)],
                    out_hbm.at[c, pl.ds(rbase, ROWS_PER_TILE)])


@functools.partial(
    pl.kernel,
    out_type=jax.ShapeDtypeStruct((NC, NP, D), jnp.float32),
    mesh=_mesh,
    scratch_types=[
        pltpu.VMEM((CH,), jnp.int32),
        pltpu.VMEM((CH,), jnp.int32),
        pltpu.VMEM((CH, D), jnp.float32),
        pltpu.VMEM_SHARED((NP, D), jnp.float32),
        pltpu.SemaphoreType.DMA,
    ],
)
def _sc_scatter(hs_hbm, src_hbm, dst_hbm, zrows_hbm, out_hbm,
                sidx, didx, rows, agg_sh, gsem):
    c = lax.axis_index("c")
    s = lax.axis_index("s")
    wid = s * NC + c
    ebase = wid * EPW
    rbase = s * ROWS_PER_TILE

    for k in range(ROWS_PER_TILE // CH):
        pltpu.sync_copy(zrows_hbm, agg_sh.at[pl.ds(rbase + k * CH, CH)])
    plsc.subcore_barrier()

    def step(i, carry):
        base = ebase + i * CH
        pltpu.sync_copy(src_hbm.at[pl.ds(base, CH)], sidx)
        pltpu.sync_copy(dst_hbm.at[pl.ds(base, CH)], didx)
        pltpu.async_copy(hs_hbm.at[sidx], rows, gsem).wait()
        pltpu.sync_copy(rows, agg_sh.at[didx], add=True)
        return carry

    lax.fori_loop(0, NCHUNK, step, 0)
    plsc.subcore_barrier()
    pltpu.sync_copy(agg_sh.at[pl.ds(rbase, ROWS_PER_TILE)],
                    out_hbm.at[c, pl.ds(rbase, ROWS_PER_TILE)])


BLK = 1024
NB = NP // BLK


def _dinv_block(dp):
    return lax.rsqrt(1.0 + dp[0] + dp[1])


def _tc1_body(dp_ref, x_ref, w_ref, o_ref):
    dinv = _dinv_block(dp_ref[...])
    o_ref[...] = dinv * jnp.dot(x_ref[...], w_ref[...],
                                preferred_element_type=jnp.float32)


def _tc_layer_body(dp_ref, g0_ref, g1_ref, hs_ref, b_ref, w_ref, o_ref):
    dinv = _dinv_block(dp_ref[...])
    t = dinv * (g0_ref[...] + g1_ref[...] + hs_ref[...]) + b_ref[...]
    t = jnp.maximum(t, 0.0)
    o_ref[...] = dinv * jnp.dot(t, w_ref[...],
                                preferred_element_type=jnp.float32)


def _tc_head_body(dp_ref, g0_ref, g1_ref, hs_ref, b_ref,
                  wd1_ref, bd1_ref, wd2_ref, bd2_ref, wo_ref, bo_ref,
                  o_ref, acc_ref):
    i = pl.program_id(0)

    @pl.when(i == 0)
    def _():
        acc_ref[...] = jnp.full_like(acc_ref, -jnp.inf)

    dinv = _dinv_block(dp_ref[...])
    t = dinv * (g0_ref[...] + g1_ref[...] + hs_ref[...]) + b_ref[...]
    t = jnp.maximum(t, 0.0)
    rows = i * BLK + lax.broadcasted_iota(jnp.int32, (BLK, D), 0)
    t = jnp.where(rows < N, t, -jnp.inf)
    acc_ref[...] = jnp.maximum(acc_ref[...],
                               jnp.max(t, axis=0, keepdims=True))

    @pl.when(i == NB - 1)
    def _():
        m = acc_ref[0:1, :]
        a = jnp.maximum(jnp.dot(m, wd1_ref[...],
                                preferred_element_type=jnp.float32)
                        + bd1_ref[...], 0.0)
        a = jnp.maximum(jnp.dot(a, wd2_ref[...],
                                preferred_element_type=jnp.float32)
                        + bd2_ref[...], 0.0)
        lg = jnp.dot(a, wo_ref[...],
                     preferred_element_type=jnp.float32) + bo_ref[...]
        e = jnp.exp(lg - jnp.max(lg, axis=-1, keepdims=True))
        o_ref[...] = e / jnp.sum(e, axis=-1, keepdims=True)


def _row_spec():
    return pl.BlockSpec((BLK, D), lambda i: (i, 0))


def _dp_spec():
    return pl.BlockSpec((2, BLK, 1), lambda i: (0, i, 0))


def _full_spec(shape):
    nd = len(shape)
    return pl.BlockSpec(shape, lambda i: (0,) * nd)


def _tc1(dp, xp, w1):
    return pl.pallas_call(
        _tc1_body,
        grid=(NB,),
        in_specs=[_dp_spec(), _row_spec(), _full_spec((D, D))],
        out_specs=_row_spec(),
        out_shape=jax.ShapeDtypeStruct((NP, D), jnp.float32),
    )(dp, xp, w1)


def _tc_layer(dp, g0, g1, hs, b, w):
    return pl.pallas_call(
        _tc_layer_body,
        grid=(NB,),
        in_specs=[_dp_spec(), _row_spec(), _row_spec(), _row_spec(),
                  _full_spec((1, D)), _full_spec((D, D))],
        out_specs=_row_spec(),
        out_shape=jax.ShapeDtypeStruct((NP, D), jnp.float32),
    )(dp, g0, g1, hs, b, w)


def _tc_head(dp, g0, g1, hs, b, wd1, bd1, wd2, bd2, wo, bo):
    du1 = wd1.shape[1]
    nl = wo.shape[1]
    return pl.pallas_call(
        _tc_head_body,
        grid=(NB,),
        in_specs=[_dp_spec(), _row_spec(), _row_spec(), _row_spec(),
                  _full_spec((1, D)),
                  _full_spec((D, du1)), _full_spec((1, du1)),
                  _full_spec((du1, D)), _full_spec((1, D)),
                  _full_spec((D, nl)), _full_spec((1, nl))],
        out_specs=_full_spec((1, nl)),
        out_shape=jax.ShapeDtypeStruct((1, nl), jnp.float32),
        scratch_shapes=[pltpu.VMEM((1, D), jnp.float32)],
    )(dp, g0, g1, hs, b, wd1, bd1, wd2, bd2, wo, bo)


def kernel(x, edge_index, W1, b1, W2, b2, W3, b3, Wd1, bd1, Wd2, bd2, Wo, bo):
    pad_rows = (N + (jnp.arange(EP - E, dtype=jnp.int32) % (NP - N)))
    src = jnp.concatenate([edge_index[0], pad_rows])
    dst = jnp.concatenate([edge_index[1], pad_rows])
    xp = jnp.pad(x, ((0, NP - N), (0, 0)))
    ones_col = jnp.ones((CH, 1), jnp.float32)
    zeros_col = jnp.zeros((ROWS_PER_TILE, 1), jnp.float32)
    zrows = jnp.zeros((CH, D), jnp.float32)
    b1r, b2r, b3r = b1.reshape(1, D), b2.reshape(1, D), b3.reshape(1, D)

    dp = _sc_degree(dst, ones_col, zeros_col)

    hs = _tc1(dp, xp, W1)
    g = _sc_scatter(hs, src, dst, zrows)
    hs = _tc_layer(dp, g[0], g[1], hs, b1r, W2)
    g = _sc_scatter(hs, src, dst, zrows)
    hs = _tc_layer(dp, g[0], g[1], hs, b2r, W3)
    g = _sc_scatter(hs, src, dst, zrows)
    return _tc_head(dp, g[0], g[1], hs, b3r, Wd1, bd1.reshape(1, -1),
                    Wd2, bd2.reshape(1, -1), Wo, bo.reshape(1, -1))


# speedup vs baseline: 21.9047x; 1.6838x over previous
import functools

import jax
import jax.numpy as jnp
from jax import lax
from jax.experimental import pallas as pl
from jax.experimental.pallas import tpu as pltpu
from jax.experimental.pallas import tpu_sc as plsc

N = 10000
NP = 10240
D = 128
E = 320000
NC, NS = 2, 16
NW = NC * NS
CH = 128
EP = NW * 80 * CH
EPW = EP // NW
NCHUNK = EPW // CH
HD = D // NC
NCH2 = EP // (NS * CH)

_mesh = plsc.VectorSubcoreMesh(
    core_axis_name="c", subcore_axis_name="s", num_cores=NC, num_subcores=NS)

ROWS_PER_TILE = NP // NS


@functools.partial(
    pl.kernel,
    out_type=jax.ShapeDtypeStruct((NC, NP, 1), jnp.float32),
    mesh=_mesh,
    scratch_types=[
        pltpu.VMEM((NCHUNK, CH), jnp.int32),
        pltpu.VMEM((CH, 1), jnp.float32),
        pltpu.VMEM_SHARED((NP, 1), jnp.float32),
        pltpu.SemaphoreType.DMA,
    ],
)
def _sc_degree(dst_hbm, ones_hbm, zeros_hbm, out_hbm, didx, ones_v, deg_sh,
               ssem):
    c = lax.axis_index("c")
    s = lax.axis_index("s")
    wid = s * NC + c
    rbase = s * ROWS_PER_TILE

    pltpu.sync_copy(ones_hbm, ones_v)
    pltpu.sync_copy(dst_hbm.at[wid], didx)
    pltpu.sync_copy(zeros_hbm, deg_sh.at[pl.ds(rbase, ROWS_PER_TILE)])
    plsc.subcore_barrier()

    def step(i, carry):
        pltpu.sync_copy(ones_v, deg_sh.at[didx.at[i]], add=True)
        return carry

    lax.fori_loop(0, NCHUNK, step, 0)
    plsc.subcore_barrier()
    pltpu.sync_copy(deg_sh.at[pl.ds(rbase, ROWS_PER_TILE)],
                    out_hbm.at[c, pl.ds(rbase, ROWS_PER_TILE)])


@functools.partial(
    pl.kernel,
    out_type=jax.ShapeDtypeStruct((NC, NP, D), jnp.float32),
    mesh=_mesh,
    scratch_types=[
        pltpu.VMEM((NCHUNK // 2, CH), jnp.int32),
        pltpu.VMEM((NCHUNK // 2, CH), jnp.int32),
        pltpu.VMEM((CH, D), jnp.float32),
        pltpu.VMEM((CH, D), jnp.float32),
        pltpu.VMEM_SHARED((NP, D), jnp.float32),
        pltpu.SemaphoreType.DMA,
    ],
)
def _sc_scatter(hs_hbm, src_hbm, dst_hbm, zrows_hbm, out_hbm,
                sidx, didx, rows_a, rows_b, agg_sh, sem):
    c = lax.axis_index("c")
    s = lax.axis_index("s")
    wid = s * NC + c
    rbase = s * ROWS_PER_TILE
    half = NCHUNK // 2

    for k in range(ROWS_PER_TILE // CH):
        pltpu.sync_copy(zrows_hbm, agg_sh.at[pl.ds(rbase + k * CH, CH)])
    plsc.subcore_barrier()

    for ph in range(2):
        pltpu.sync_copy(src_hbm.at[wid, pl.ds(ph * half, half)], sidx)
        pltpu.sync_copy(dst_hbm.at[wid, pl.ds(ph * half, half)], didx)

        def step(j, carry):
            i = 2 * j
            ca = pltpu.async_copy(hs_hbm.at[sidx.at[i]], rows_a, sem)
            cb = pltpu.async_copy(hs_hbm.at[sidx.at[i + 1]], rows_b, sem)
            ca.wait()
            pltpu.sync_copy(rows_a, agg_sh.at[didx.at[i]], add=True)
            cb.wait()
            pltpu.sync_copy(rows_b, agg_sh.at[didx.at[i + 1]], add=True)
            return carry

        lax.fori_loop(0, half // 2, step, 0)
    plsc.subcore_barrier()
    pltpu.sync_copy(agg_sh.at[pl.ds(rbase, ROWS_PER_TILE)],
                    out_hbm.at[c, pl.ds(rbase, ROWS_PER_TILE)])


BLK = 1024
NB = NP // BLK


def _dinv_block(dp):
    return lax.rsqrt(1.0 + dp[0] + dp[1])


def _tc1_body(dp_ref, x_ref, w_ref, o_ref):
    dinv = _dinv_block(dp_ref[...])
    o_ref[...] = dinv * jnp.dot(x_ref[...], w_ref[...],
                                preferred_element_type=jnp.float32)


def _tc_layer_body(dp_ref, g0_ref, g1_ref, hs_ref, b_ref, w_ref, o_ref):
    dinv = _dinv_block(dp_ref[...])
    t = dinv * (g0_ref[...] + g1_ref[...] + hs_ref[...]) + b_ref[...]
    t = jnp.maximum(t, 0.0)
    o_ref[...] = dinv * jnp.dot(t, w_ref[...],
                                preferred_element_type=jnp.float32)


def _tc_head_body(dp_ref, g0_ref, g1_ref, hs_ref, b_ref,
                  wd1_ref, bd1_ref, wd2_ref, bd2_ref, wo_ref, bo_ref,
                  o_ref, acc_ref):
    i = pl.program_id(0)

    @pl.when(i == 0)
    def _():
        acc_ref[...] = jnp.full_like(acc_ref, -jnp.inf)

    dinv = _dinv_block(dp_ref[...])
    t = dinv * (g0_ref[...] + g1_ref[...] + hs_ref[...]) + b_ref[...]
    t = jnp.maximum(t, 0.0)
    rows = i * BLK + lax.broadcasted_iota(jnp.int32, (BLK, D), 0)
    t = jnp.where(rows < N, t, -jnp.inf)
    acc_ref[...] = jnp.maximum(acc_ref[...],
                               jnp.max(t, axis=0, keepdims=True))

    @pl.when(i == NB - 1)
    def _():
        m = acc_ref[0:1, :]
        a = jnp.maximum(jnp.dot(m, wd1_ref[...],
                                preferred_element_type=jnp.float32)
                        + bd1_ref[...], 0.0)
        a = jnp.maximum(jnp.dot(a, wd2_ref[...],
                                preferred_element_type=jnp.float32)
                        + bd2_ref[...], 0.0)
        lg = jnp.dot(a, wo_ref[...],
                     preferred_element_type=jnp.float32) + bo_ref[...]
        e = jnp.exp(lg - jnp.max(lg, axis=-1, keepdims=True))
        o_ref[...] = e / jnp.sum(e, axis=-1, keepdims=True)


def _row_spec():
    return pl.BlockSpec((BLK, D), lambda i: (i, 0))


def _dp_spec():
    return pl.BlockSpec((2, BLK, 1), lambda i: (0, i, 0))


def _full_spec(shape):
    nd = len(shape)
    return pl.BlockSpec(shape, lambda i: (0,) * nd)


def _tc1(dp, xp, w1):
    return pl.pallas_call(
        _tc1_body,
        grid=(NB,),
        in_specs=[_dp_spec(), _row_spec(), _full_spec((D, D))],
        out_specs=_row_spec(),
        out_shape=jax.ShapeDtypeStruct((NP, D), jnp.float32),
    )(dp, xp, w1)


def _tc_layer(dp, g0, g1, hs, b, w):
    return pl.pallas_call(
        _tc_layer_body,
        grid=(NB,),
        in_specs=[_dp_spec(), _row_spec(), _row_spec(), _row_spec(),
                  _full_spec((1, D)), _full_spec((D, D))],
        out_specs=_row_spec(),
        out_shape=jax.ShapeDtypeStruct((NP, D), jnp.float32),
    )(dp, g0, g1, hs, b, w)


def _tc_head(dp, g0, g1, hs, b, wd1, bd1, wd2, bd2, wo, bo):
    du1 = wd1.shape[1]
    nl = wo.shape[1]
    return pl.pallas_call(
        _tc_head_body,
        grid=(NB,),
        in_specs=[_dp_spec(), _row_spec(), _row_spec(), _row_spec(),
                  _full_spec((1, D)),
                  _full_spec((D, du1)), _full_spec((1, du1)),
                  _full_spec((du1, D)), _full_spec((1, D)),
                  _full_spec((D, nl)), _full_spec((1, nl))],
        out_specs=_full_spec((1, nl)),
        out_shape=jax.ShapeDtypeStruct((1, nl), jnp.float32),
        scratch_shapes=[pltpu.VMEM((1, D), jnp.float32)],
    )(dp, g0, g1, hs, b, wd1, bd1, wd2, bd2, wo, bo)


def kernel(x, edge_index, W1, b1, W2, b2, W3, b3, Wd1, bd1, Wd2, bd2, Wo, bo):
    pad_rows = (N + (jnp.arange(EP - E, dtype=jnp.int32) % (NP - N)))
    src = jnp.concatenate([edge_index[0], pad_rows])
    dst = jnp.concatenate([edge_index[1], pad_rows])
    dst_deg = dst.reshape(NW, NCHUNK, CH)
    src_sc = src.reshape(NW, NCHUNK, CH)
    dst_sc = dst_deg
    xp = jnp.pad(x, ((0, NP - N), (0, 0)))
    ones_col = jnp.ones((CH, 1), jnp.float32)
    zeros_col = jnp.zeros((ROWS_PER_TILE, 1), jnp.float32)
    zrows = jnp.zeros((CH, D), jnp.float32)
    b1r, b2r, b3r = b1.reshape(1, D), b2.reshape(1, D), b3.reshape(1, D)

    dp = _sc_degree(dst_deg, ones_col, zeros_col)

    hs = _tc1(dp, xp, W1)
    g = _sc_scatter(hs, src_sc, dst_sc, zrows)
    hs = _tc_layer(dp, g[0], g[1], hs, b1r, W2)
    g = _sc_scatter(hs, src_sc, dst_sc, zrows)
    hs = _tc_layer(dp, g[0], g[1], hs, b2r, W3)
    g = _sc_scatter(hs, src_sc, dst_sc, zrows)
    return _tc_head(dp, g[0], g[1], hs, b3r, Wd1, bd1.reshape(1, -1),
                    Wd2, bd2.reshape(1, -1), Wo, bo.reshape(1, -1))


# speedup vs baseline: 26.0629x; 1.1898x over previous
import functools

import jax
import jax.numpy as jnp
from jax import lax
from jax.experimental import pallas as pl
from jax.experimental.pallas import tpu as pltpu
from jax.experimental.pallas import tpu_sc as plsc

N = 10000
NP = 10240
D = 128
E = 320000
NC, NS = 2, 16
NW = NC * NS
CH = 128
EP = NW * 80 * CH
EPW = EP // NW
NCHUNK = EPW // CH
CH2 = 64
NCK = EPW // CH2
NPH = 4
QCK = NCK // NPH

_mesh = plsc.VectorSubcoreMesh(
    core_axis_name="c", subcore_axis_name="s", num_cores=NC, num_subcores=NS)

ROWS_PER_TILE = NP // NS


@functools.partial(
    pl.kernel,
    out_type=jax.ShapeDtypeStruct((NC, NP, 1), jnp.float32),
    mesh=_mesh,
    scratch_types=[
        pltpu.VMEM((NCHUNK, CH), jnp.int32),
        pltpu.VMEM((CH, 1), jnp.float32),
        pltpu.VMEM_SHARED((NP, 1), jnp.float32),
        pltpu.SemaphoreType.DMA,
    ],
)
def _sc_degree(dst_hbm, ones_hbm, zeros_hbm, out_hbm, didx, ones_v, deg_sh,
               ssem):
    c = lax.axis_index("c")
    s = lax.axis_index("s")
    wid = s * NC + c
    rbase = s * ROWS_PER_TILE

    pltpu.sync_copy(ones_hbm, ones_v)
    pltpu.sync_copy(dst_hbm.at[wid], didx)
    pltpu.sync_copy(zeros_hbm, deg_sh.at[pl.ds(rbase, ROWS_PER_TILE)])
    plsc.subcore_barrier()

    def step(i, carry):
        pltpu.async_copy(ones_v, deg_sh.at[didx.at[i]], ssem, add=True)
        return carry

    lax.fori_loop(0, NCHUNK, step, 0)

    def drain(i, carry):
        pltpu.make_async_copy(ones_v, deg_sh.at[didx.at[0]], ssem).wait()
        return carry

    lax.fori_loop(0, NCHUNK, drain, 0)
    plsc.subcore_barrier()
    pltpu.sync_copy(deg_sh.at[pl.ds(rbase, ROWS_PER_TILE)],
                    out_hbm.at[c, pl.ds(rbase, ROWS_PER_TILE)])


@functools.partial(
    pl.kernel,
    out_type=jax.ShapeDtypeStruct((NC, NP, D), jnp.float32),
    mesh=_mesh,
    scratch_types=[
        pltpu.VMEM((QCK + 3, CH2), jnp.int32),
        pltpu.VMEM((QCK, CH2), jnp.int32),
        pltpu.VMEM((CH2, D), jnp.float32),
        pltpu.VMEM((CH2, D), jnp.float32),
        pltpu.VMEM((CH2, D), jnp.float32),
        pltpu.VMEM((CH2, D), jnp.float32),
        pltpu.VMEM_SHARED((NP, D), jnp.float32),
        pltpu.SemaphoreType.DMA,
        pltpu.SemaphoreType.DMA,
    ],
)
def _sc_scatter(hs_hbm, src_hbm, dst_hbm, zrows_hbm, out_hbm,
                sidx, didx, r0, r1, r2, r3, agg_sh, gsem, ssem):
    c = lax.axis_index("c")
    s = lax.axis_index("s")
    wid = s * NC + c
    rbase = s * ROWS_PER_TILE
    bufs = (r0, r1, r2, r3)

    for k in range(ROWS_PER_TILE // CH):
        pltpu.sync_copy(zrows_hbm, agg_sh.at[pl.ds(rbase + k * CH, CH)])
    plsc.subcore_barrier()

    def gather(i, buf):
        return pltpu.async_copy(hs_hbm.at[sidx.at[i]], buf, gsem)

    def gather_wait(buf):
        pltpu.make_async_copy(hs_hbm.at[sidx.at[0]], buf, gsem).wait()

    def scat(i, buf):
        return pltpu.async_copy(buf, agg_sh.at[didx.at[i]], ssem, add=True)

    def scat_wait(buf):
        pltpu.make_async_copy(buf, agg_sh.at[didx.at[0]], ssem).wait()

    for ph in range(NPH):
        pltpu.sync_copy(src_hbm.at[wid, ph], sidx)
        pltpu.sync_copy(dst_hbm.at[wid, ph], didx)
        gather(0, r0)
        gather(1, r1)
        gather(2, r2)

        def step(j, carry):
            for k in range(4):
                i = 4 * j + k
                gather_wait(bufs[k])
                scat(i, bufs[k])
                if k == 0:
                    @pl.when(j > 0)
                    def _():
                        scat_wait(bufs[3])
                else:
                    scat_wait(bufs[k - 1])
                gather(i + 3, bufs[(k + 3) % 4])
            return carry

        lax.fori_loop(0, QCK // 4, step, 0)
        gather_wait(r0)
        gather_wait(r1)
        gather_wait(r2)
        scat_wait(r3)
    plsc.subcore_barrier()
    pltpu.sync_copy(agg_sh.at[pl.ds(rbase, ROWS_PER_TILE)],
                    out_hbm.at[c, pl.ds(rbase, ROWS_PER_TILE)])


BLK = 1024
NB = NP // BLK


def _dinv_block(dp):
    return lax.rsqrt(1.0 + dp[0] + dp[1])


def _tc1_body(dp_ref, x_ref, w_ref, o_ref):
    dinv = _dinv_block(dp_ref[...])
    o_ref[...] = dinv * jnp.dot(x_ref[...], w_ref[...],
                                preferred_element_type=jnp.float32)


def _tc_layer_body(dp_ref, g0_ref, g1_ref, hs_ref, b_ref, w_ref, o_ref):
    dinv = _dinv_block(dp_ref[...])
    t = dinv * (g0_ref[...] + g1_ref[...] + hs_ref[...]) + b_ref[...]
    t = jnp.maximum(t, 0.0)
    o_ref[...] = dinv * jnp.dot(t, w_ref[...],
                                preferred_element_type=jnp.float32)


def _tc_head_body(dp_ref, g0_ref, g1_ref, hs_ref, b_ref,
                  wd1_ref, bd1_ref, wd2_ref, bd2_ref, wo_ref, bo_ref,
                  o_ref, acc_ref):
    i = pl.program_id(0)

    @pl.when(i == 0)
    def _():
        acc_ref[...] = jnp.full_like(acc_ref, -jnp.inf)

    dinv = _dinv_block(dp_ref[...])
    t = dinv * (g0_ref[...] + g1_ref[...] + hs_ref[...]) + b_ref[...]
    t = jnp.maximum(t, 0.0)
    rows = i * BLK + lax.broadcasted_iota(jnp.int32, (BLK, D), 0)
    t = jnp.where(rows < N, t, -jnp.inf)
    acc_ref[...] = jnp.maximum(acc_ref[...],
                               jnp.max(t, axis=0, keepdims=True))

    @pl.when(i == NB - 1)
    def _():
        m = acc_ref[0:1, :]
        a = jnp.maximum(jnp.dot(m, wd1_ref[...],
                                preferred_element_type=jnp.float32)
                        + bd1_ref[...], 0.0)
        a = jnp.maximum(jnp.dot(a, wd2_ref[...],
                                preferred_element_type=jnp.float32)
                        + bd2_ref[...], 0.0)
        lg = jnp.dot(a, wo_ref[...],
                     preferred_element_type=jnp.float32) + bo_ref[...]
        e = jnp.exp(lg - jnp.max(lg, axis=-1, keepdims=True))
        o_ref[...] = e / jnp.sum(e, axis=-1, keepdims=True)


def _row_spec():
    return pl.BlockSpec((BLK, D), lambda i: (i, 0))


def _dp_spec():
    return pl.BlockSpec((2, BLK, 1), lambda i: (0, i, 0))


def _full_spec(shape):
    nd = len(shape)
    return pl.BlockSpec(shape, lambda i: (0,) * nd)


def _tc1(dp, xp, w1):
    return pl.pallas_call(
        _tc1_body,
        grid=(NB,),
        in_specs=[_dp_spec(), _row_spec(), _full_spec((D, D))],
        out_specs=_row_spec(),
        out_shape=jax.ShapeDtypeStruct((NP, D), jnp.float32),
    )(dp, xp, w1)


def _tc_layer(dp, g0, g1, hs, b, w):
    return pl.pallas_call(
        _tc_layer_body,
        grid=(NB,),
        in_specs=[_dp_spec(), _row_spec(), _row_spec(), _row_spec(),
                  _full_spec((1, D)), _full_spec((D, D))],
        out_specs=_row_spec(),
        out_shape=jax.ShapeDtypeStruct((NP, D), jnp.float32),
    )(dp, g0, g1, hs, b, w)


def _tc_head(dp, g0, g1, hs, b, wd1, bd1, wd2, bd2, wo, bo):
    du1 = wd1.shape[1]
    nl = wo.shape[1]
    return pl.pallas_call(
        _tc_head_body,
        grid=(NB,),
        in_specs=[_dp_spec(), _row_spec(), _row_spec(), _row_spec(),
                  _full_spec((1, D)),
                  _full_spec((D, du1)), _full_spec((1, du1)),
                  _full_spec((du1, D)), _full_spec((1, D)),
                  _full_spec((D, nl)), _full_spec((1, nl))],
        out_specs=_full_spec((1, nl)),
        out_shape=jax.ShapeDtypeStruct((1, nl), jnp.float32),
        scratch_shapes=[pltpu.VMEM((1, D), jnp.float32)],
    )(dp, g0, g1, hs, b, wd1, bd1, wd2, bd2, wo, bo)


def kernel(x, edge_index, W1, b1, W2, b2, W3, b3, Wd1, bd1, Wd2, bd2, Wo, bo):
    pad_rows = (N + (jnp.arange(EP - E, dtype=jnp.int32) % (NP - N)))
    src = jnp.concatenate([edge_index[0], pad_rows])
    dst = jnp.concatenate([edge_index[1], pad_rows])
    dst_deg = dst.reshape(NW, NCHUNK, CH)
    srcr = src.reshape(NW, NCK, CH2)
    dstr = dst.reshape(NW, NCK, CH2)
    src_sc = jnp.stack(
        [jnp.concatenate([srcr[:, q * QCK:(q + 1) * QCK],
                          srcr[:, (q + 1) * QCK - 3:(q + 1) * QCK]], axis=1)
         for q in range(NPH)], axis=1)
    dst_sc = jnp.stack([dstr[:, q * QCK:(q + 1) * QCK]
                        for q in range(NPH)], axis=1)
    xp = jnp.pad(x, ((0, NP - N), (0, 0)))
    ones_col = jnp.ones((CH, 1), jnp.float32)
    zeros_col = jnp.zeros((ROWS_PER_TILE, 1), jnp.float32)
    zrows = jnp.zeros((CH, D), jnp.float32)
    b1r, b2r, b3r = b1.reshape(1, D), b2.reshape(1, D), b3.reshape(1, D)

    dp = _sc_degree(dst_deg, ones_col, zeros_col)

    hs = _tc1(dp, xp, W1)
    g = _sc_scatter(hs, src_sc, dst_sc, zrows)
    hs = _tc_layer(dp, g[0], g[1], hs, b1r, W2)
    g = _sc_scatter(hs, src_sc, dst_sc, zrows)
    hs = _tc_layer(dp, g[0], g[1], hs, b2r, W3)
    g = _sc_scatter(hs, src_sc, dst_sc, zrows)
    return _tc_head(dp, g[0], g[1], hs, b3r, Wd1, bd1.reshape(1, -1),
                    Wd2, bd2.reshape(1, -1), Wo, bo.reshape(1, -1))
